# baseline (speedup 1.0000x reference)
import functools

import jax
import jax.numpy as jnp
import numpy as np
from jax import lax
from jax.experimental import pallas as pl
from jax.experimental.pallas import tpu as pltpu
from jax.experimental.pallas import tpu_sc as plsc

B = 8
N = 2048
C_FEAT = 64
NPOINT = 512
NSAMPLE = 32
MLP_OUT = 128
R2 = np.float32(np.float64(0.2) ** 2)

_rng = np.random.default_rng(42)
_STARTS = np.array([int(_rng.integers(0, N)) for _ in range(B)], dtype=np.int32)


def _fps_body(starts_ref, xyz24_ref, idx_ref, nx_ref, ny_ref, nz_ref):
    XYZ = xyz24_ref[...]
    X = XYZ[0:8]
    Y = XYZ[8:16]
    Z = XYZ[16:24]
    iota_n24 = lax.broadcasted_iota(jnp.int32, (3 * B, N), 1)
    iota_p = lax.broadcasted_iota(jnp.int32, (B, NPOINT), 1)

    dist0 = jnp.full((B, N), 1e10, dtype=jnp.float32)
    far0 = starts_ref[...]
    oI0 = jnp.zeros((B, NPOINT), dtype=jnp.int32)
    oX0 = jnp.zeros((B, NPOINT), dtype=jnp.float32)
    oY0 = jnp.zeros((B, NPOINT), dtype=jnp.float32)
    oZ0 = jnp.zeros((B, NPOINT), dtype=jnp.float32)

    def body(i, st):
        dist, far, oI, oX, oY, oZ = st
        far24 = jnp.concatenate([far, far, far], axis=0)
        s24 = jnp.sum(jnp.where(iota_n24 == far24, XYZ, 0.0),
                      axis=1, keepdims=True)
        sx = s24[0:8]
        sy = s24[8:16]
        sz = s24[16:24]
        here = iota_p == i
        oI = jnp.where(here, far, oI)
        oX = jnp.where(here, sx, oX)
        oY = jnp.where(here, sy, oY)
        oZ = jnp.where(here, sz, oZ)
        dx = X - sx
        dy = Y - sy
        dz = Z - sz
        d = dx * dx + dy * dy + dz * dz
        dist = jnp.minimum(dist, d)
        far = jnp.argmax(dist, axis=1, keepdims=True).astype(jnp.int32)
        return dist, far, oI, oX, oY, oZ

    _, _, oI, oX, oY, oZ = lax.fori_loop(
        0, NPOINT, body, (dist0, far0, oI0, oX0, oY0, oZ0), unroll=2)
    idx_ref[...] = oI
    nx_ref[...] = oX
    ny_ref[...] = oY
    nz_ref[...] = oZ


def _run_fps(xc, yc, zc, interpret=False):
    xyz24 = jnp.concatenate([xc, yc, zc], axis=0)
    return pl.pallas_call(
        _fps_body,
        out_shape=(
            jax.ShapeDtypeStruct((B, NPOINT), jnp.int32),
            jax.ShapeDtypeStruct((B, NPOINT), jnp.float32),
            jax.ShapeDtypeStruct((B, NPOINT), jnp.float32),
            jax.ShapeDtypeStruct((B, NPOINT), jnp.float32),
        ),
        interpret=interpret,
    )(jnp.asarray(_STARTS.reshape(B, 1)), xyz24)


_CB = 128


def _ce_net(n, sort_full):
    prs = []

    def merge(lo, m, d):
        if m > 1:
            h = m // 2
            for i in range(lo, lo + h):
                prs.append((i, i + h, d))
            merge(lo, h, d)
            merge(lo + h, h, d)

    def srt(lo, m, d):
        if m > 1:
            h = m // 2
            srt(lo, h, True)
            srt(lo + h, h, False)
            merge(lo, m, d)

    if sort_full:
        srt(0, n, True)
    else:
        merge(0, n, True)
    return prs


_SORT32 = _ce_net(NSAMPLE, True)
_MERGE32 = _ce_net(NSAMPLE, False)

_QBITS = 19
_QOUT = (1 << _QBITS) - 1
_QSCALE = np.float32((_QOUT - 1) / R2)
_QBIAS = 0x20000000


def _apply_net(a, net):
    for (i, j, asc) in net:
        lo = jnp.minimum(a[i], a[j])
        hi = jnp.maximum(a[i], a[j])
        a[i], a[j] = (lo, hi) if asc else (hi, lo)
    return a


def _topk_body(b0, x32_ref, y32_ref, z32_ref, nx_ref, ny_ref, nz_ref,
               out_ref, vscr):
    b = pl.program_id(0) + b0
    cxb = jnp.broadcast_to(nx_ref[0, 0], (8, _CB))
    cyb = jnp.broadcast_to(ny_ref[0, 0], (8, _CB))
    czb = jnp.broadcast_to(nz_ref[0, 0], (8, _CB))
    isub = lax.broadcasted_iota(jnp.int32, (8, _CB), 0) * NSAMPLE

    def build(k, _):
        Xk = x32_ref[0, pl.ds(k * 8, 8), :]
        Yk = y32_ref[0, pl.ds(k * 8, 8), :]
        Zk = z32_ref[0, pl.ds(k * 8, 8), :]
        a = []
        for s in range(NSAMPLE):
            xd = jnp.broadcast_to(Xk[:, s:s + 1], (8, _CB)) - cxb
            yd = jnp.broadcast_to(Yk[:, s:s + 1], (8, _CB)) - cyb
            zd = jnp.broadcast_to(Zk[:, s:s + 1], (8, _CB)) - czb
            d = xd * xd + yd * yd + zd * zd
            qi = jnp.minimum((d * _QSCALE).astype(jnp.int32), _QOUT - 1)
            q = jnp.where(d <= R2, qi, _QOUT)
            idx = isub + (k * 8 * NSAMPLE + s)
            pk = (lax.shift_left(q, 11) | idx) + _QBIAS
            a.append(lax.bitcast_convert_type(pk, jnp.float32))
        a = _apply_net(a, _SORT32)
        for s in range(NSAMPLE):
            vscr[k, s] = a[s]
        return 0

    lax.fori_loop(0, 8, build, 0)

    for h in (4, 2, 1):
        for k in range(h):
            a = [jnp.minimum(vscr[k, s], vscr[k + h, NSAMPLE - 1 - s])
                 for s in range(NSAMPLE)]
            a = _apply_net(a, _MERGE32)
            for s in range(NSAMPLE):
                vscr[k, s] = a[s]

    a = [vscr[0, s] for s in range(NSAMPLE)]
    for h in (4, 2, 1):
        a = [jnp.minimum(a[s], pltpu.roll(a[NSAMPLE - 1 - s], 8 - h, 0))
             for s in range(NSAMPLE)]
        a = _apply_net(a, _MERGE32)

    for s in range(NSAMPLE):
        pk = lax.bitcast_convert_type(a[s], jnp.int32)
        gi = (pk & 0x7FF) + b * N
        out_ref[0, 0, pl.ds(s, 1), :] = gi[0:1, :]


def _run_topk(xc, yc, zc, nx, ny, nz, b0=0, interpret=False):
    nb = xc.shape[0]
    nblk = NPOINT // _CB
    ng = N // NSAMPLE
    return pl.pallas_call(
        functools.partial(_topk_body, b0),
        grid=(nb, nblk),
        in_specs=[
            pl.BlockSpec((1, ng, NSAMPLE), lambda b, j: (b, 0, 0)),
            pl.BlockSpec((1, ng, NSAMPLE), lambda b, j: (b, 0, 0)),
            pl.BlockSpec((1, ng, NSAMPLE), lambda b, j: (b, 0, 0)),
            pl.BlockSpec((1, 1, 1, _CB), lambda b, j: (b, j, 0, 0)),
            pl.BlockSpec((1, 1, 1, _CB), lambda b, j: (b, j, 0, 0)),
            pl.BlockSpec((1, 1, 1, _CB), lambda b, j: (b, j, 0, 0)),
        ],
        out_specs=pl.BlockSpec((1, 1, NSAMPLE, _CB), lambda b, j: (b, j, 0, 0)),
        out_shape=jax.ShapeDtypeStruct((nb, nblk, NSAMPLE, _CB), jnp.int32),
        scratch_shapes=[pltpu.VMEM((8, NSAMPLE, 8, _CB), jnp.float32)],
        interpret=interpret,
    )(xc.reshape(nb, ng, NSAMPLE), yc.reshape(nb, ng, NSAMPLE),
      zc.reshape(nb, ng, NSAMPLE),
      nx.reshape(nb, nblk, 1, _CB), ny.reshape(nb, nblk, 1, _CB),
      nz.reshape(nb, nblk, 1, _CB))


_ROWS = B * NPOINT * NSAMPLE
_NW = 32
_CH = 512
_D = 128


def _sc_gather_call(gidx, featx, nxp, nyp, nzp):
    rows = gidx.shape[0]
    ncent = nxp.shape[0]
    rpw = rows // _NW
    nchunk = rpw // _CH
    mesh = plsc.VectorSubcoreMesh(core_axis_name="c", subcore_axis_name="s")

    @functools.partial(
        pl.kernel,
        out_type=jax.ShapeDtypeStruct((rows, _D), jnp.float32),
        mesh=mesh,
        compiler_params=pltpu.CompilerParams(
            needs_layout_passes=False, use_tc_tiling_on_sc=False),
        scratch_types=(
            pltpu.VMEM((_CH,), jnp.int32),
            pltpu.VMEM((_CH, _D), jnp.float32),
            pltpu.VMEM((ncent,), jnp.float32),
            pltpu.VMEM((ncent,), jnp.float32),
            pltpu.VMEM((ncent,), jnp.float32),
            pltpu.SemaphoreType.DMA,
        ),
    )
    def k(gidx_h, featx_h, nxp_h, nyp_h, nzp_h, out_h,
          idx_v, fbuf, nxv, nyv, nzv, sem):
        wid = lax.axis_index("s") * 2 + lax.axis_index("c")
        pltpu.sync_copy(nxp_h, nxv)
        pltpu.sync_copy(nyp_h, nyv)
        pltpu.sync_copy(nzp_h, nzv)

        iota16 = lax.iota(jnp.int32, 16)
        mx = (iota16 == 0).astype(jnp.float32)
        my = (iota16 == 1).astype(jnp.float32)
        mz = (iota16 == 2).astype(jnp.float32)

        for c in range(nchunk):
            r0 = wid * rpw + c * _CH
            pltpu.sync_copy(gidx_h.at[pl.ds(r0, _CH)], idx_v)
            pltpu.async_copy(featx_h.at[idx_v], fbuf, sem).wait()

            def gbody(g, _):
                cid = r0 // NSAMPLE + g
                cidv = jnp.broadcast_to(cid, (16,)).astype(jnp.int32)
                gx = plsc.load_gather(nxv, [cidv])
                gy = plsc.load_gather(nyv, [cidv])
                gz = plsc.load_gather(nzv, [cidv])
                v = gx * mx + gy * my + gz * mz

                def rbody(j, _):
                    row = g * NSAMPLE + j
                    fbuf[row, pl.ds(C_FEAT, 16)] = (
                        fbuf[row, pl.ds(C_FEAT, 16)] - v)
                    return 0

                lax.fori_loop(0, NSAMPLE, rbody, 0)
                return 0

            lax.fori_loop(0, _CH // NSAMPLE, gbody, 0)
            pltpu.sync_copy(fbuf, out_h.at[pl.ds(r0, _CH)])

    return k(gidx, featx, nxp, nyp, nzp)


_RB = 2048
_GB = _RB // NSAMPLE


def _mlp_body(xall_ref, g_ref, wcat_ref, b1_ref, w2_ref, b2_ref,
              wa1f_ref, ba1_ref, wa2_ref, ba2_ref, out_ref):
    Xall = xall_ref[...]
    G = g_ref[...]
    Wcat = wcat_ref[...]
    b1 = b1_ref[...]
    W2 = w2_ref[...]
    b2 = b2_ref[...]
    Wa1f = wa1f_ref[...]
    ba1 = ba1_ref[...]
    Wa2 = wa2_ref[...]
    ba2 = ba2_ref[...]

    dot = functools.partial(jnp.dot, preferred_element_type=jnp.float32)
    Z = dot(Xall, Wcat)
    h = jax.nn.relu(Z[:, :MLP_OUT] + b1)
    xa = Z[:, MLP_OUT:]
    fp = jax.nn.relu(dot(h, W2) + b2)
    mean = dot(G, fp) * np.float32(1.0 / NSAMPLE)
    A = dot(fp, Wa1f) + xa + ba1
    A3 = A.reshape(_GB, NSAMPLE, MLP_OUT) - dot(mean, Wa1f)[:, None, :]
    hw = jax.nn.relu(A3).reshape(_RB, MLP_OUT)
    alpha = jax.nn.sigmoid(dot(hw, Wa2) + ba2)
    f_out = dot(G, alpha * fp)
    out_ref[...] = f_out


def _run_mlp(xall, gmat, wcat, b1, w2, b2, wa1f, ba1, wa2, ba2,
             interpret=False):
    rows = xall.shape[0]
    nblk = rows // _RB
    full = lambda r, c: pl.BlockSpec((r, c), lambda i: (0, 0))
    return pl.pallas_call(
        _mlp_body,
        grid=(nblk,),
        in_specs=[
            pl.BlockSpec((_RB, _D), lambda i: (i, 0)),
            full(_GB, _RB),
            full(_D, 2 * MLP_OUT),
            full(1, MLP_OUT),
            full(MLP_OUT, MLP_OUT),
            full(1, MLP_OUT),
            full(MLP_OUT, MLP_OUT),
            full(1, MLP_OUT),
            full(MLP_OUT, MLP_OUT),
            full(1, MLP_OUT),
        ],
        out_specs=pl.BlockSpec((_GB, MLP_OUT), lambda i: (i, 0)),
        out_shape=jax.ShapeDtypeStruct((rows // NSAMPLE, MLP_OUT), jnp.float32),
        interpret=interpret,
    )(xall, gmat, wcat, b1, w2, b2, wa1f, ba1, wa2, ba2)


def kernel(xyz, features, W1, b1, W2, b2, Wa1, ba1, Wa2, ba2):
    xc = xyz[:, :, 0]
    yc = xyz[:, :, 1]
    zc = xyz[:, :, 2]

    fps_idx, nx, ny, nz = _run_fps(xc, yc, zc)
    new_xyz = jnp.stack([nx, ny, nz], axis=-1)

    featx = jnp.concatenate(
        [features.reshape(B * N, C_FEAT), xyz.reshape(B * N, 3),
         jnp.zeros((B * N, _D - C_FEAT - 3), jnp.float32)], axis=1)

    w1p = jnp.zeros((_D, MLP_OUT), W1.dtype)
    w1p = w1p.at[:C_FEAT, :].set(W1[3:, :]).at[C_FEAT:C_FEAT + 3, :].set(W1[:3, :])
    wa1xp = jnp.zeros((_D, MLP_OUT), Wa1.dtype)
    wa1xp = wa1xp.at[C_FEAT:C_FEAT + 3, :].set(Wa1[:3, :])
    wcat = jnp.concatenate([w1p, wa1xp], axis=1)
    wa1f = Wa1[3:, :]
    gmat = jnp.repeat(jnp.eye(_GB, dtype=jnp.float32), NSAMPLE, axis=1)

    hb = B // 2
    outs = []
    for half in range(2):
        sl = slice(half * hb, (half + 1) * hb)
        gidx4 = _run_topk(xc[sl], yc[sl], zc[sl], nx[sl], ny[sl], nz[sl],
                          b0=half * hb)
        gidx = gidx4.transpose(0, 1, 3, 2).reshape(hb * NPOINT * NSAMPLE)
        xall = _sc_gather_call(gidx, featx, nx[sl].reshape(hb * NPOINT),
                               ny[sl].reshape(hb * NPOINT),
                               nz[sl].reshape(hb * NPOINT))
        outs.append(_run_mlp(xall, gmat, wcat, b1.reshape(1, -1), W2,
                             b2.reshape(1, -1), wa1f, ba1.reshape(1, -1),
                             Wa2, ba2.reshape(1, -1)))
    f_out = jnp.concatenate(outs, axis=0)
    return new_xyz, f_out.reshape(B, NPOINT, MLP_OUT)


# speedup vs baseline: 1.0308x; 1.0308x over previous
import functools

import jax
import jax.numpy as jnp
import numpy as np
from jax import lax
from jax.experimental import pallas as pl
from jax.experimental.pallas import tpu as pltpu
from jax.experimental.pallas import tpu_sc as plsc

B = 8
N = 2048
C_FEAT = 64
NPOINT = 512
NSAMPLE = 32
MLP_OUT = 128
R2 = np.float32(np.float64(0.2) ** 2)

_rng = np.random.default_rng(42)
_STARTS = np.array([int(_rng.integers(0, N)) for _ in range(B)], dtype=np.int32)


def _fps_body(starts_ref, xyz24_ref, idx_ref, nx_ref, ny_ref, nz_ref):
    XYZ = xyz24_ref[...]
    X = XYZ[0:8]
    Y = XYZ[8:16]
    Z = XYZ[16:24]
    iota_n24 = lax.broadcasted_iota(jnp.int32, (3 * B, N), 1)
    iota_p = lax.broadcasted_iota(jnp.int32, (B, NPOINT), 1)

    dist0 = jnp.full((B, N), 1e10, dtype=jnp.float32)
    far0 = starts_ref[...]
    oI0 = jnp.zeros((B, NPOINT), dtype=jnp.int32)
    oX0 = jnp.zeros((B, NPOINT), dtype=jnp.float32)
    oY0 = jnp.zeros((B, NPOINT), dtype=jnp.float32)
    oZ0 = jnp.zeros((B, NPOINT), dtype=jnp.float32)

    def body(i, st):
        dist, far, oI, oX, oY, oZ = st
        far24 = jnp.concatenate([far, far, far], axis=0)
        s24 = jnp.sum(jnp.where(iota_n24 == far24, XYZ, 0.0),
                      axis=1, keepdims=True)
        sx = s24[0:8]
        sy = s24[8:16]
        sz = s24[16:24]
        here = iota_p == i
        oI = jnp.where(here, far, oI)
        oX = jnp.where(here, sx, oX)
        oY = jnp.where(here, sy, oY)
        oZ = jnp.where(here, sz, oZ)
        dx = X - sx
        dy = Y - sy
        dz = Z - sz
        d = dx * dx + dy * dy + dz * dz
        dist = jnp.minimum(dist, d)
        far = jnp.argmax(dist, axis=1, keepdims=True).astype(jnp.int32)
        return dist, far, oI, oX, oY, oZ

    _, _, oI, oX, oY, oZ = lax.fori_loop(
        0, NPOINT, body, (dist0, far0, oI0, oX0, oY0, oZ0), unroll=2)
    idx_ref[...] = oI
    nx_ref[...] = oX
    ny_ref[...] = oY
    nz_ref[...] = oZ


def _run_fps(xc, yc, zc, interpret=False):
    xyz24 = jnp.concatenate([xc, yc, zc], axis=0)
    return pl.pallas_call(
        _fps_body,
        out_shape=(
            jax.ShapeDtypeStruct((B, NPOINT), jnp.int32),
            jax.ShapeDtypeStruct((B, NPOINT), jnp.float32),
            jax.ShapeDtypeStruct((B, NPOINT), jnp.float32),
            jax.ShapeDtypeStruct((B, NPOINT), jnp.float32),
        ),
        interpret=interpret,
    )(jnp.asarray(_STARTS.reshape(B, 1)), xyz24)


_CB = 128


def _ce_net(n, sort_full):
    prs = []

    def merge(lo, m, d):
        if m > 1:
            h = m // 2
            for i in range(lo, lo + h):
                prs.append((i, i + h, d))
            merge(lo, h, d)
            merge(lo + h, h, d)

    def srt(lo, m, d):
        if m > 1:
            h = m // 2
            srt(lo, h, True)
            srt(lo + h, h, False)
            merge(lo, m, d)

    if sort_full:
        srt(0, n, True)
    else:
        merge(0, n, True)
    return prs


_SORT32 = _ce_net(NSAMPLE, True)
_MERGE32 = _ce_net(NSAMPLE, False)

_QBITS = 19
_QOUT = (1 << _QBITS) - 1
_QSCALE = np.float32((_QOUT - 1) / R2)
_QBIAS = 0x20000000


def _apply_net(a, net):
    for (i, j, asc) in net:
        lo = jnp.minimum(a[i], a[j])
        hi = jnp.maximum(a[i], a[j])
        a[i], a[j] = (lo, hi) if asc else (hi, lo)
    return a


def _topk_body(b0, x32_ref, y32_ref, z32_ref, nx_ref, ny_ref, nz_ref,
               out_ref, vscr):
    b = pl.program_id(0) + b0
    cxb = jnp.broadcast_to(nx_ref[0, 0], (8, _CB))
    cyb = jnp.broadcast_to(ny_ref[0, 0], (8, _CB))
    czb = jnp.broadcast_to(nz_ref[0, 0], (8, _CB))
    isub = lax.broadcasted_iota(jnp.int32, (8, _CB), 0) * NSAMPLE

    def build(k, _):
        Xk = x32_ref[0, pl.ds(k * 8, 8), :]
        Yk = y32_ref[0, pl.ds(k * 8, 8), :]
        Zk = z32_ref[0, pl.ds(k * 8, 8), :]
        a = []
        for s in range(NSAMPLE):
            xd = jnp.broadcast_to(Xk[:, s:s + 1], (8, _CB)) - cxb
            yd = jnp.broadcast_to(Yk[:, s:s + 1], (8, _CB)) - cyb
            zd = jnp.broadcast_to(Zk[:, s:s + 1], (8, _CB)) - czb
            d = xd * xd + yd * yd + zd * zd
            qi = jnp.minimum((d * _QSCALE).astype(jnp.int32), _QOUT - 1)
            q = jnp.where(d <= R2, qi, _QOUT)
            idx = isub + (k * 8 * NSAMPLE + s)
            pk = (lax.shift_left(q, 11) | idx) + _QBIAS
            a.append(lax.bitcast_convert_type(pk, jnp.float32))
        a = _apply_net(a, _SORT32)
        for s in range(NSAMPLE):
            vscr[k, s] = a[s]
        return 0

    lax.fori_loop(0, 8, build, 0)

    for h in (4, 2, 1):
        for k in range(h):
            a = [jnp.minimum(vscr[k, s], vscr[k + h, NSAMPLE - 1 - s])
                 for s in range(NSAMPLE)]
            a = _apply_net(a, _MERGE32)
            for s in range(NSAMPLE):
                vscr[k, s] = a[s]

    a = [vscr[0, s] for s in range(NSAMPLE)]
    for h in (4, 2, 1):
        a = [jnp.minimum(a[s], pltpu.roll(a[NSAMPLE - 1 - s], 8 - h, 0))
             for s in range(NSAMPLE)]
        a = _apply_net(a, _MERGE32)

    for s in range(NSAMPLE):
        pk = lax.bitcast_convert_type(a[s], jnp.int32)
        gi = (pk & 0x7FF) + b * N
        out_ref[0, 0, pl.ds(s, 1), :] = gi[0:1, :]


def _run_topk(xc, yc, zc, nx, ny, nz, b0=0, interpret=False):
    nb = xc.shape[0]
    nblk = NPOINT // _CB
    ng = N // NSAMPLE
    return pl.pallas_call(
        functools.partial(_topk_body, b0),
        grid=(nb, nblk),
        in_specs=[
            pl.BlockSpec((1, ng, NSAMPLE), lambda b, j: (b, 0, 0)),
            pl.BlockSpec((1, ng, NSAMPLE), lambda b, j: (b, 0, 0)),
            pl.BlockSpec((1, ng, NSAMPLE), lambda b, j: (b, 0, 0)),
            pl.BlockSpec((1, 1, 1, _CB), lambda b, j: (b, j, 0, 0)),
            pl.BlockSpec((1, 1, 1, _CB), lambda b, j: (b, j, 0, 0)),
            pl.BlockSpec((1, 1, 1, _CB), lambda b, j: (b, j, 0, 0)),
        ],
        out_specs=pl.BlockSpec((1, 1, NSAMPLE, _CB), lambda b, j: (b, j, 0, 0)),
        out_shape=jax.ShapeDtypeStruct((nb, nblk, NSAMPLE, _CB), jnp.int32),
        scratch_shapes=[pltpu.VMEM((8, NSAMPLE, 8, _CB), jnp.float32)],
        interpret=interpret,
    )(xc.reshape(nb, ng, NSAMPLE), yc.reshape(nb, ng, NSAMPLE),
      zc.reshape(nb, ng, NSAMPLE),
      nx.reshape(nb, nblk, 1, _CB), ny.reshape(nb, nblk, 1, _CB),
      nz.reshape(nb, nblk, 1, _CB))


_ROWS = B * NPOINT * NSAMPLE
_NW = 32
_CH = 512
_D = 128


def _sc_gather_call(gidx, featx, nxp, nyp, nzp):
    rows = gidx.shape[0]
    ncent = nxp.shape[0]
    rpw = rows // _NW
    nchunk = rpw // _CH
    mesh = plsc.VectorSubcoreMesh(core_axis_name="c", subcore_axis_name="s")

    @functools.partial(
        pl.kernel,
        out_type=jax.ShapeDtypeStruct((rows, _D), jnp.float32),
        mesh=mesh,
        compiler_params=pltpu.CompilerParams(
            needs_layout_passes=False, use_tc_tiling_on_sc=False),
        scratch_types=(
            pltpu.VMEM((_CH,), jnp.int32),
            pltpu.VMEM((_CH, _D), jnp.float32),
            pltpu.VMEM((ncent,), jnp.float32),
            pltpu.VMEM((ncent,), jnp.float32),
            pltpu.VMEM((ncent,), jnp.float32),
            pltpu.SemaphoreType.DMA,
        ),
    )
    def k(gidx_h, featx_h, nxp_h, nyp_h, nzp_h, out_h,
          idx_v, fbuf, nxv, nyv, nzv, sem):
        wid = lax.axis_index("s") * 2 + lax.axis_index("c")
        pltpu.sync_copy(nxp_h, nxv)
        pltpu.sync_copy(nyp_h, nyv)
        pltpu.sync_copy(nzp_h, nzv)

        iota16 = lax.iota(jnp.int32, 16)
        mx = (iota16 == 0).astype(jnp.float32)
        my = (iota16 == 1).astype(jnp.float32)
        mz = (iota16 == 2).astype(jnp.float32)

        for c in range(nchunk):
            r0 = wid * rpw + c * _CH
            pltpu.sync_copy(gidx_h.at[pl.ds(r0, _CH)], idx_v)
            pltpu.async_copy(featx_h.at[idx_v], fbuf, sem).wait()

            def gbody(g, _):
                cid = r0 // NSAMPLE + g
                cidv = jnp.broadcast_to(cid, (16,)).astype(jnp.int32)
                gx = plsc.load_gather(nxv, [cidv])
                gy = plsc.load_gather(nyv, [cidv])
                gz = plsc.load_gather(nzv, [cidv])
                v = gx * mx + gy * my + gz * mz

                def rbody(j, _):
                    row = g * NSAMPLE + j
                    fbuf[row, pl.ds(C_FEAT, 16)] = (
                        fbuf[row, pl.ds(C_FEAT, 16)] - v)
                    return 0

                lax.fori_loop(0, NSAMPLE, rbody, 0)
                return 0

            lax.fori_loop(0, _CH // NSAMPLE, gbody, 0)
            pltpu.sync_copy(fbuf, out_h.at[pl.ds(r0, _CH)])

    return k(gidx, featx, nxp, nyp, nzp)


_RB = 2048
_GB = _RB // NSAMPLE


def _mlp_body(xall_ref, wcat_ref, b1_ref, w2_ref, b2_ref,
              wa1f_ref, ba1_ref, wa2_ref, ba2_ref, out_ref):
    Xall = xall_ref[...]
    Wcat = wcat_ref[...]
    b1 = b1_ref[...]
    W2 = w2_ref[...]
    b2 = b2_ref[...]
    Wa1f = wa1f_ref[...]
    ba1 = ba1_ref[...]
    Wa2 = wa2_ref[...]
    ba2 = ba2_ref[...]

    dot = functools.partial(jnp.dot, preferred_element_type=jnp.float32)
    Z = dot(Xall, Wcat)
    h = jax.nn.relu(Z[:, :MLP_OUT] + b1)
    xa = Z[:, MLP_OUT:]
    fp = jax.nn.relu(dot(h, W2) + b2)
    fp3 = fp.reshape(_GB, NSAMPLE, MLP_OUT)
    mean = jnp.mean(fp3, axis=1)
    A = dot(fp, Wa1f) + xa + ba1
    A3 = A.reshape(_GB, NSAMPLE, MLP_OUT) - dot(mean, Wa1f)[:, None, :]
    hw = jax.nn.relu(A3).reshape(_RB, MLP_OUT)
    alpha = jax.nn.sigmoid(dot(hw, Wa2) + ba2)
    f_out = jnp.sum(alpha.reshape(_GB, NSAMPLE, MLP_OUT) * fp3, axis=1)
    out_ref[...] = f_out


def _run_mlp(xall, wcat, b1, w2, b2, wa1f, ba1, wa2, ba2, interpret=False):
    rows = xall.shape[0]
    nblk = rows // _RB
    full = lambda r, c: pl.BlockSpec((r, c), lambda i: (0, 0))
    return pl.pallas_call(
        _mlp_body,
        grid=(nblk,),
        in_specs=[
            pl.BlockSpec((_RB, _D), lambda i: (i, 0)),
            full(_D, 2 * MLP_OUT),
            full(1, MLP_OUT),
            full(MLP_OUT, MLP_OUT),
            full(1, MLP_OUT),
            full(MLP_OUT, MLP_OUT),
            full(1, MLP_OUT),
            full(MLP_OUT, MLP_OUT),
            full(1, MLP_OUT),
        ],
        out_specs=pl.BlockSpec((_GB, MLP_OUT), lambda i: (i, 0)),
        out_shape=jax.ShapeDtypeStruct((rows // NSAMPLE, MLP_OUT), jnp.float32),
        interpret=interpret,
    )(xall, wcat, b1, w2, b2, wa1f, ba1, wa2, ba2)


def kernel(xyz, features, W1, b1, W2, b2, Wa1, ba1, Wa2, ba2):
    xc = xyz[:, :, 0]
    yc = xyz[:, :, 1]
    zc = xyz[:, :, 2]

    fps_idx, nx, ny, nz = _run_fps(xc, yc, zc)
    new_xyz = jnp.stack([nx, ny, nz], axis=-1)

    featx = jnp.concatenate(
        [features.reshape(B * N, C_FEAT), xyz.reshape(B * N, 3),
         jnp.zeros((B * N, _D - C_FEAT - 3), jnp.float32)], axis=1)

    w1p = jnp.zeros((_D, MLP_OUT), W1.dtype)
    w1p = w1p.at[:C_FEAT, :].set(W1[3:, :]).at[C_FEAT:C_FEAT + 3, :].set(W1[:3, :])
    wa1xp = jnp.zeros((_D, MLP_OUT), Wa1.dtype)
    wa1xp = wa1xp.at[C_FEAT:C_FEAT + 3, :].set(Wa1[:3, :])
    wcat = jnp.concatenate([w1p, wa1xp], axis=1)
    wa1f = Wa1[3:, :]

    hb = B // 2
    outs = []
    for half in range(2):
        sl = slice(half * hb, (half + 1) * hb)
        gidx4 = _run_topk(xc[sl], yc[sl], zc[sl], nx[sl], ny[sl], nz[sl],
                          b0=half * hb)
        gidx = gidx4.transpose(0, 1, 3, 2).reshape(hb * NPOINT * NSAMPLE)
        xall = _sc_gather_call(gidx, featx, nx[sl].reshape(hb * NPOINT),
                               ny[sl].reshape(hb * NPOINT),
                               nz[sl].reshape(hb * NPOINT))
        outs.append(_run_mlp(xall, wcat, b1.reshape(1, -1), W2,
                             b2.reshape(1, -1), wa1f, ba1.reshape(1, -1),
                             Wa2, ba2.reshape(1, -1)))
    f_out = jnp.concatenate(outs, axis=0)
    return new_xyz, f_out.reshape(B, NPOINT, MLP_OUT)
